# baseline (speedup 1.0000x reference)
import functools

import jax
import jax.numpy as jnp
from jax import lax
from jax.experimental import pallas as pl
from jax.experimental.pallas import tpu as pltpu
from jax.experimental.pallas import tpu_sc as plsc

D = 768
H = 12
DH = 64
E = 8
FF = 3072
S = 2048
TM = 256
NT = S // TM

TMS = 128
PAD = 5120
NTILES = PAD // TMS
NW = 32


def _rms(x, w):
    return x * jax.lax.rsqrt(jnp.mean(x * x, axis=-1, keepdims=True) + 1e-6) * w


def _qkv_body(x_ref, w1_ref, wq_ref, wk_ref, wv_ref, bq_ref, bk_ref, bv_ref,
              cos_ref, sin_ref, q_ref, k_ref, v_ref):
    x = x_ref[...]
    h = _rms(x, w1_ref[...])
    q = jnp.dot(h, wq_ref[...], preferred_element_type=jnp.float32) + bq_ref[...]
    k = jnp.dot(h, wk_ref[...], preferred_element_type=jnp.float32) + bk_ref[...]
    v = jnp.dot(h, wv_ref[...], preferred_element_type=jnp.float32) + bv_ref[...]
    cos = cos_ref[...]
    sin = sin_ref[...]
    lane = jax.lax.broadcasted_iota(jnp.int32, (TM, D), 1)
    lo = (lane % DH) < (DH // 2)
    z = jnp.zeros((TM, DH // 2), jnp.float32)

    def rot(t):
        tl = jnp.concatenate([t[:, DH // 2:], z], axis=1)
        tr = jnp.concatenate([z, t[:, :D - DH // 2]], axis=1)
        return jnp.where(lo, -tl, tr)

    q_ref[...] = (q * cos + rot(q) * sin) * 0.125
    k_ref[...] = k * cos + rot(k) * sin
    v_ref[...] = v


def _qkv(x2, norm1_w, Wq, Wk, Wv, bq, bk, bv, cosf, sinf):
    full = pl.BlockSpec((D, D), lambda i: (0, 0))
    row = pl.BlockSpec((1, D), lambda i: (0, 0))
    tile = pl.BlockSpec((TM, D), lambda i: (i, 0))
    return pl.pallas_call(
        _qkv_body,
        grid=(NT,),
        in_specs=[tile, row, full, full, full, row, row, row, tile, tile],
        out_specs=[tile, tile, tile],
        out_shape=[jax.ShapeDtypeStruct((S, D), jnp.float32)] * 3,
    )(x2, norm1_w, Wq, Wk, Wv, bq, bk, bv, cosf, sinf)


def _attn_body(q_ref, k_ref, v_ref, o_ref):
    i = pl.program_id(1)

    def flat(w):
        rowi = i * TM + jax.lax.broadcasted_iota(jnp.int32, (TM, w), 0)
        coli = jax.lax.broadcasted_iota(jnp.int32, (TM, w), 1)
        outs = []
        for lo in (0, DH):
            q = q_ref[:, lo:lo + DH]
            kt = k_ref[0:w, lo:lo + DH]
            vt = v_ref[0:w, lo:lo + DH]
            s = jax.lax.dot_general(q, kt, (((1,), (1,)), ((), ())),
                                    preferred_element_type=jnp.float32)
            s = jnp.where(coli <= rowi, s, -1e9)
            m = jnp.max(s, axis=1, keepdims=True)
            p = jnp.exp(s - m)
            l = jnp.sum(p, axis=1, keepdims=True)
            outs.append(jnp.dot(p, vt,
                                preferred_element_type=jnp.float32) / l)
        o_ref[...] = jnp.concatenate(outs, axis=1)

    for b in range(4):
        @pl.when(jnp.logical_and(i >= 2 * b, i < 2 * b + 2))
        def _(w=512 * (b + 1)):
            flat(w)


def _attn(q, k, v):
    return pl.pallas_call(
        _attn_body,
        grid=(H // 2, NT),
        in_specs=[
            pl.BlockSpec((TM, 2 * DH), lambda hp, i: (i, hp)),
            pl.BlockSpec((S, 2 * DH), lambda hp, i: (0, hp)),
            pl.BlockSpec((S, 2 * DH), lambda hp, i: (0, hp)),
        ],
        out_specs=pl.BlockSpec((TM, 2 * DH), lambda hp, i: (i, hp)),
        out_shape=jax.ShapeDtypeStruct((S, D), jnp.float32),
    )(q, k, v)


def _post_body(o_ref, x_ref, wo_ref, bo_ref, w2n_ref, wr_ref, br_ref,
               h1_ref, hn_ref, rl_ref, rank_ref, tw_ref, cnt_ref, carry_ref):
    i = pl.program_id(0)
    att = jnp.dot(o_ref[...], wo_ref[...],
                  preferred_element_type=jnp.float32) + bo_ref[...]
    h1 = x_ref[...] + att
    h1_ref[...] = h1
    hn = _rms(h1, w2n_ref[...])
    hn_ref[...] = hn
    rl = jnp.dot(hn, wr_ref[...], preferred_element_type=jnp.float32) + br_ref[...]
    rl_ref[...] = rl
    ii = jax.lax.broadcasted_iota(jnp.int32, (TM, E), 1)
    m1 = jnp.max(rl, axis=1, keepdims=True)
    i1 = jnp.min(jnp.where(rl == m1, ii, E), axis=1, keepdims=True)
    ml = jnp.where(ii == i1, -1e30, rl)
    m2 = jnp.max(ml, axis=1, keepdims=True)
    i2 = jnp.min(jnp.where(ml == m2, ii, E), axis=1, keepdims=True)
    e2 = jnp.exp(m2 - m1)
    w1 = 1.0 / (1.0 + e2)
    w2 = e2 / (1.0 + e2)
    tw_ref[...] = jnp.concatenate([w1, w2], axis=1)

    oh1 = (ii == i1).astype(jnp.float32)
    oh2 = (ii == i2).astype(jnp.float32)
    both = oh1 + oh2
    tri = (jax.lax.broadcasted_iota(jnp.int32, (TM, TM), 0) >
           jax.lax.broadcasted_iota(jnp.int32, (TM, TM), 1)).astype(jnp.float32)
    pref = jnp.dot(tri, both, preferred_element_type=jnp.float32)

    @pl.when(i == 0)
    def _():
        carry_ref[...] = jnp.zeros_like(carry_ref)

    base = carry_ref[...]
    p = pref + base
    r0 = jnp.sum(oh1 * p, axis=1, keepdims=True)
    r1 = jnp.sum(oh2 * (p + oh1), axis=1, keepdims=True)
    enc0 = r0.astype(jnp.int32) + i1 * 65536
    enc1 = r1.astype(jnp.int32) + i2 * 65536
    rank_ref[...] = jnp.concatenate([enc0, enc1], axis=1)
    carry_ref[...] = base + jnp.sum(both, axis=0, keepdims=True)
    cnt_ref[...] = carry_ref[...]


def _post(o768, x2, Wo, bo, norm2_w, Wr, br):
    tile = pl.BlockSpec((TM, D), lambda i: (i, 0))
    return pl.pallas_call(
        _post_body,
        grid=(NT,),
        in_specs=[
            tile, tile,
            pl.BlockSpec((D, D), lambda i: (0, 0)),
            pl.BlockSpec((1, D), lambda i: (0, 0)),
            pl.BlockSpec((1, D), lambda i: (0, 0)),
            pl.BlockSpec((D, E), lambda i: (0, 0)),
            pl.BlockSpec((1, E), lambda i: (0, 0)),
        ],
        out_specs=[
            tile, tile,
            pl.BlockSpec((TM, E), lambda i: (i, 0)),
            pl.BlockSpec((TM, 2), lambda i: (i, 0)),
            pl.BlockSpec((TM, 2), lambda i: (i, 0)),
            pl.BlockSpec((1, E), lambda i: (0, 0)),
        ],
        out_shape=[
            jax.ShapeDtypeStruct((S, D), jnp.float32),
            jax.ShapeDtypeStruct((S, D), jnp.float32),
            jax.ShapeDtypeStruct((S, E), jnp.float32),
            jax.ShapeDtypeStruct((S, 2), jnp.int32),
            jax.ShapeDtypeStruct((S, 2), jnp.float32),
            jax.ShapeDtypeStruct((1, E), jnp.float32),
        ],
        scratch_shapes=[pltpu.VMEM((1, E), jnp.float32)],
    )(o768, x2, Wo, bo, norm2_w, Wr, br)


def _sc_gather(table, idx, nrows):
    bpw = nrows // NW
    nch = 4
    ch = bpw // nch
    nbuf = 3
    mesh = plsc.VectorSubcoreMesh(core_axis_name="c", subcore_axis_name="s")

    @functools.partial(
        pl.kernel,
        out_type=jax.ShapeDtypeStruct((nrows, D), jnp.float32),
        mesh=mesh,
        scratch_types=[
            pltpu.VMEM((nch, ch), jnp.int32),
            pltpu.VMEM((nbuf, ch, D), jnp.float32),
            pltpu.SemaphoreType.DMA,
            pltpu.SemaphoreType.DMA,
        ],
    )
    def k(table_hbm, idx_hbm, out_hbm, idx_v, rows_v, gsem, wsem):
        wid = lax.axis_index("s") * 2 + lax.axis_index("c")
        pltpu.sync_copy(idx_hbm.at[wid], idx_v)
        g = [None] * nch
        w = [None] * nch
        for j in range(min(nbuf, nch)):
            g[j] = pltpu.async_copy(table_hbm.at[idx_v.at[j]],
                                    rows_v.at[j % nbuf], gsem)
        for j in range(nch):
            g[j].wait()
            w[j] = pltpu.async_copy(
                rows_v.at[j % nbuf],
                out_hbm.at[pl.ds(wid * bpw + j * ch, ch)], wsem)
            nxt = j + nbuf
            if nxt < nch:
                w[j].wait()
                g[nxt] = pltpu.async_copy(table_hbm.at[idx_v.at[nxt]],
                                          rows_v.at[nxt % nbuf], gsem)
        for j in range(nch):
            if w[j] is not None and (j + nbuf >= nch):
                w[j].wait()

    return k(table, idx.reshape(NW, nch, ch))


def _gemm_body(te_ref, nv_ref, g_ref, w1_ref, b1_ref, w2_ref, b2_ref,
               out_ref):
    t = pl.program_id(0)

    @pl.when(t < nv_ref[0])
    def _():
        g = jnp.dot(g_ref[...].astype(jnp.bfloat16),
                    w1_ref[0].astype(jnp.bfloat16),
                    preferred_element_type=jnp.float32) + b1_ref[0]
        g = 0.5 * g * (1.0 + jax.lax.erf(g * (2.0 ** -0.5)))
        oe = jnp.dot(g.astype(jnp.bfloat16),
                     w2_ref[0].astype(jnp.bfloat16),
                     preferred_element_type=jnp.float32) + b2_ref[0]
        out_ref[...] = oe


def _grouped_gemm(gathered, tile_e, nvalid, W1, b1, W2, b2):
    grid_spec = pltpu.PrefetchScalarGridSpec(
        num_scalar_prefetch=2,
        grid=(NTILES,),
        in_specs=[
            pl.BlockSpec((TMS, D), lambda t, te, nv: (t, 0)),
            pl.BlockSpec((1, D, FF), lambda t, te, nv: (te[t], 0, 0)),
            pl.BlockSpec((1, 1, FF), lambda t, te, nv: (te[t], 0, 0)),
            pl.BlockSpec((1, FF, D), lambda t, te, nv: (te[t], 0, 0)),
            pl.BlockSpec((1, 1, D), lambda t, te, nv: (te[t], 0, 0)),
        ],
        out_specs=pl.BlockSpec((TMS, D), lambda t, te, nv: (t, 0)),
    )
    return pl.pallas_call(
        _gemm_body,
        grid_spec=grid_spec,
        out_shape=jax.ShapeDtypeStruct((PAD, D), jnp.float32),
        compiler_params=pltpu.CompilerParams(
            dimension_semantics=("arbitrary",)),
    )(tile_e, nvalid, gathered, W1.reshape(E, D, FF), b1.reshape(E, 1, FF),
      W2.reshape(E, FF, D), b2.reshape(E, 1, D))


def _comb_body(h1_ref, g0_ref, g1_ref, tw_ref, out_ref):
    tw = tw_ref[...]
    out_ref[...] = (h1_ref[...] + g0_ref[...] * tw[:, 0:1]
                    + g1_ref[...] * tw[:, 1:2])


def _combine(h1, g0, g1, tw):
    tile = pl.BlockSpec((TM, D), lambda i: (i, 0))
    return pl.pallas_call(
        _comb_body,
        grid=(NT,),
        in_specs=[tile, tile, tile, pl.BlockSpec((TM, 2), lambda i: (i, 0))],
        out_specs=tile,
        out_shape=jax.ShapeDtypeStruct((S, D), jnp.float32),
    )(h1, g0, g1, tw)


def _sc_dispatch(hn, sl0, sl1):
    bpw = S // NW
    nch = 2
    ch = bpw // nch
    mesh = plsc.VectorSubcoreMesh(core_axis_name="c", subcore_axis_name="s")

    @functools.partial(
        pl.kernel,
        out_type=jax.ShapeDtypeStruct((PAD, D), jnp.float32),
        mesh=mesh,
        scratch_types=[
            pltpu.VMEM((nch, ch), jnp.int32),
            pltpu.VMEM((nch, ch), jnp.int32),
            pltpu.VMEM((nch, ch, D), jnp.float32),
            pltpu.SemaphoreType.DMA,
            pltpu.SemaphoreType.DMA,
        ],
    )
    def k(hn_hbm, sl0_hbm, sl1_hbm, out_hbm, i0_v, i1_v, rows_v, lsem, ssem):
        wid = lax.axis_index("s") * 2 + lax.axis_index("c")
        pltpu.sync_copy(sl0_hbm.at[wid], i0_v)
        pltpu.sync_copy(sl1_hbm.at[wid], i1_v)
        ld = [None] * nch
        st = [None] * (2 * nch)
        for j in range(nch):
            ld[j] = pltpu.async_copy(
                hn_hbm.at[pl.ds(wid * bpw + j * ch, ch)], rows_v.at[j], lsem)
        for j in range(nch):
            ld[j].wait()
            st[2 * j] = pltpu.async_copy(rows_v.at[j],
                                         out_hbm.at[i0_v.at[j]], ssem)
            st[2 * j + 1] = pltpu.async_copy(rows_v.at[j],
                                             out_hbm.at[i1_v.at[j]], ssem)
        for s_ in st:
            s_.wait()

    return k(hn, sl0.reshape(NW, nch, ch), sl1.reshape(NW, nch, ch))


def _dispatch_indices(rank_enc, counts):
    cnt = counts[0]
    pc = jnp.ceil(cnt / TMS) * TMS
    po = (jnp.cumsum(pc) - pc).astype(jnp.int32)
    ends = (jnp.cumsum(pc)).astype(jnp.int32)
    exp = rank_enc // 65536
    rnk = rank_enc - exp * 65536
    slot = rnk
    for e in range(E):
        slot = slot + jnp.where(exp == e, po[e], 0)
    tile_e = jnp.clip(
        jnp.sum((ends[None, :] <= (jnp.arange(NTILES, dtype=jnp.int32)
                                   * TMS)[:, None]).astype(jnp.int32), axis=1),
        0, E - 1).astype(jnp.int32)
    nvalid = (ends[E - 1] // TMS).reshape(1)
    return slot, tile_e, nvalid


def kernel(x, norm1_w, Wq, bq, Wk, bk, Wv, bv, Wo, bo, norm2_w, Wr, br,
           W1, b1, W2, b2):
    x2 = x.reshape(S, D)
    inv_freq = 1.0 / (10000.0 ** (jnp.arange(0, DH, 2, dtype=jnp.float32) / DH))
    freqs = jnp.arange(S, dtype=jnp.float32)[:, None] * inv_freq[None, :]
    emb = jnp.concatenate([freqs, freqs], axis=-1)
    cosf = jnp.tile(jnp.cos(emb), (1, H))
    sinf = jnp.tile(jnp.sin(emb), (1, H))

    q, k, v = _qkv(x2, norm1_w.reshape(1, D), Wq, Wk, Wv,
                   bq.reshape(1, D), bk.reshape(1, D), bv.reshape(1, D),
                   cosf, sinf)
    o768 = _attn(q, k, v)

    h1, hn, rl, rank_enc, tw, counts = _post(o768, x2, Wo, bo.reshape(1, D),
                                             norm2_w.reshape(1, D), Wr,
                                             br.reshape(1, E))

    slot, tile_e, nvalid = _dispatch_indices(rank_enc, counts)
    gathered = _sc_dispatch(hn, slot[:, 0], slot[:, 1])
    rows = _grouped_gemm(gathered, tile_e, nvalid, W1, b1, W2, b2)
    posf = jnp.concatenate([slot[:, 0], slot[:, 1]])
    g = _sc_gather(rows, posf, 2 * S)
    out = _combine(h1, g[:S], g[S:], tw).reshape(1, S, D)
    return out, rl


# speedup vs baseline: 1.0514x; 1.0514x over previous
import functools

import jax
import jax.numpy as jnp
from jax import lax
from jax.experimental import pallas as pl
from jax.experimental.pallas import tpu as pltpu
from jax.experimental.pallas import tpu_sc as plsc

D = 768
H = 12
DH = 64
E = 8
FF = 3072
S = 2048
TM = 256
NT = S // TM

TMS = 256
PAD = 6144
NTILES = PAD // TMS
NW = 32


def _rms(x, w):
    return x * jax.lax.rsqrt(jnp.mean(x * x, axis=-1, keepdims=True) + 1e-6) * w


def _qkv_body(x_ref, w1_ref, wq_ref, wk_ref, wv_ref, bq_ref, bk_ref, bv_ref,
              cos_ref, sin_ref, q_ref, k_ref, v_ref):
    x = x_ref[...]
    h = _rms(x, w1_ref[...])
    q = jnp.dot(h, wq_ref[...], preferred_element_type=jnp.float32) + bq_ref[...]
    k = jnp.dot(h, wk_ref[...], preferred_element_type=jnp.float32) + bk_ref[...]
    v = jnp.dot(h, wv_ref[...], preferred_element_type=jnp.float32) + bv_ref[...]
    cos = cos_ref[...]
    sin = sin_ref[...]
    lane = jax.lax.broadcasted_iota(jnp.int32, (TM, D), 1)
    lo = (lane % DH) < (DH // 2)
    z = jnp.zeros((TM, DH // 2), jnp.float32)

    def rot(t):
        tl = jnp.concatenate([t[:, DH // 2:], z], axis=1)
        tr = jnp.concatenate([z, t[:, :D - DH // 2]], axis=1)
        return jnp.where(lo, -tl, tr)

    q_ref[...] = (q * cos + rot(q) * sin) * 0.125
    k_ref[...] = k * cos + rot(k) * sin
    v_ref[...] = v


def _qkv(x2, norm1_w, Wq, Wk, Wv, bq, bk, bv, cosf, sinf):
    full = pl.BlockSpec((D, D), lambda i: (0, 0))
    row = pl.BlockSpec((1, D), lambda i: (0, 0))
    tile = pl.BlockSpec((TM, D), lambda i: (i, 0))
    return pl.pallas_call(
        _qkv_body,
        grid=(NT,),
        in_specs=[tile, row, full, full, full, row, row, row, tile, tile],
        out_specs=[tile, tile, tile],
        out_shape=[jax.ShapeDtypeStruct((S, D), jnp.float32)] * 3,
    )(x2, norm1_w, Wq, Wk, Wv, bq, bk, bv, cosf, sinf)


def _attn_body(q_ref, k_ref, v_ref, o_ref):
    i = pl.program_id(1)

    def flat(w):
        rowi = i * TM + jax.lax.broadcasted_iota(jnp.int32, (TM, w), 0)
        coli = jax.lax.broadcasted_iota(jnp.int32, (TM, w), 1)
        outs = []
        for lo in (0, DH):
            q = q_ref[:, lo:lo + DH]
            kt = k_ref[0:w, lo:lo + DH]
            vt = v_ref[0:w, lo:lo + DH]
            s = jax.lax.dot_general(q, kt, (((1,), (1,)), ((), ())),
                                    preferred_element_type=jnp.float32)
            s = jnp.where(coli <= rowi, s, -1e9)
            m = jnp.max(s, axis=1, keepdims=True)
            p = jnp.exp(s - m)
            l = jnp.sum(p, axis=1, keepdims=True)
            outs.append(jnp.dot(p, vt,
                                preferred_element_type=jnp.float32) / l)
        o_ref[...] = jnp.concatenate(outs, axis=1)

    for b in range(4):
        @pl.when(jnp.logical_and(i >= 2 * b, i < 2 * b + 2))
        def _(w=512 * (b + 1)):
            flat(w)


def _attn(q, k, v):
    return pl.pallas_call(
        _attn_body,
        grid=(H // 2, NT),
        in_specs=[
            pl.BlockSpec((TM, 2 * DH), lambda hp, i: (i, hp)),
            pl.BlockSpec((S, 2 * DH), lambda hp, i: (0, hp)),
            pl.BlockSpec((S, 2 * DH), lambda hp, i: (0, hp)),
        ],
        out_specs=pl.BlockSpec((TM, 2 * DH), lambda hp, i: (i, hp)),
        out_shape=jax.ShapeDtypeStruct((S, D), jnp.float32),
    )(q, k, v)


def _post_body(o_ref, x_ref, wo_ref, bo_ref, w2n_ref, wr_ref, br_ref,
               h1_ref, hn_ref, rl_ref, rank_ref, tw_ref, cnt_ref, carry_ref):
    i = pl.program_id(0)
    att = jnp.dot(o_ref[...], wo_ref[...],
                  preferred_element_type=jnp.float32) + bo_ref[...]
    h1 = x_ref[...] + att
    h1_ref[...] = h1
    hn = _rms(h1, w2n_ref[...])
    hn_ref[...] = hn
    rl = jnp.dot(hn, wr_ref[...], preferred_element_type=jnp.float32) + br_ref[...]
    rl_ref[...] = rl
    ii = jax.lax.broadcasted_iota(jnp.int32, (TM, E), 1)
    m1 = jnp.max(rl, axis=1, keepdims=True)
    i1 = jnp.min(jnp.where(rl == m1, ii, E), axis=1, keepdims=True)
    ml = jnp.where(ii == i1, -1e30, rl)
    m2 = jnp.max(ml, axis=1, keepdims=True)
    i2 = jnp.min(jnp.where(ml == m2, ii, E), axis=1, keepdims=True)
    e2 = jnp.exp(m2 - m1)
    w1 = 1.0 / (1.0 + e2)
    w2 = e2 / (1.0 + e2)
    tw_ref[...] = jnp.concatenate([w1, w2], axis=1)

    oh1 = (ii == i1).astype(jnp.float32)
    oh2 = (ii == i2).astype(jnp.float32)
    both = oh1 + oh2
    tri = (jax.lax.broadcasted_iota(jnp.int32, (TM, TM), 0) >
           jax.lax.broadcasted_iota(jnp.int32, (TM, TM), 1)).astype(jnp.float32)
    pref = jnp.dot(tri, both, preferred_element_type=jnp.float32)

    @pl.when(i == 0)
    def _():
        carry_ref[...] = jnp.zeros_like(carry_ref)

    base = carry_ref[...]
    p = pref + base
    r0 = jnp.sum(oh1 * p, axis=1, keepdims=True)
    r1 = jnp.sum(oh2 * (p + oh1), axis=1, keepdims=True)
    enc0 = r0.astype(jnp.int32) + i1 * 65536
    enc1 = r1.astype(jnp.int32) + i2 * 65536
    rank_ref[...] = jnp.concatenate([enc0, enc1], axis=1)
    carry_ref[...] = base + jnp.sum(both, axis=0, keepdims=True)
    cnt_ref[...] = carry_ref[...]


def _post(o768, x2, Wo, bo, norm2_w, Wr, br):
    tile = pl.BlockSpec((TM, D), lambda i: (i, 0))
    return pl.pallas_call(
        _post_body,
        grid=(NT,),
        in_specs=[
            tile, tile,
            pl.BlockSpec((D, D), lambda i: (0, 0)),
            pl.BlockSpec((1, D), lambda i: (0, 0)),
            pl.BlockSpec((1, D), lambda i: (0, 0)),
            pl.BlockSpec((D, E), lambda i: (0, 0)),
            pl.BlockSpec((1, E), lambda i: (0, 0)),
        ],
        out_specs=[
            tile, tile,
            pl.BlockSpec((TM, E), lambda i: (i, 0)),
            pl.BlockSpec((TM, 2), lambda i: (i, 0)),
            pl.BlockSpec((TM, 2), lambda i: (i, 0)),
            pl.BlockSpec((1, E), lambda i: (0, 0)),
        ],
        out_shape=[
            jax.ShapeDtypeStruct((S, D), jnp.float32),
            jax.ShapeDtypeStruct((S, D), jnp.float32),
            jax.ShapeDtypeStruct((S, E), jnp.float32),
            jax.ShapeDtypeStruct((S, 2), jnp.int32),
            jax.ShapeDtypeStruct((S, 2), jnp.float32),
            jax.ShapeDtypeStruct((1, E), jnp.float32),
        ],
        scratch_shapes=[pltpu.VMEM((1, E), jnp.float32)],
    )(o768, x2, Wo, bo, norm2_w, Wr, br)


def _sc_gather(table, idx, nrows):
    bpw = nrows // NW
    nch = 4
    ch = bpw // nch
    nbuf = 3
    mesh = plsc.VectorSubcoreMesh(core_axis_name="c", subcore_axis_name="s")

    @functools.partial(
        pl.kernel,
        out_type=jax.ShapeDtypeStruct((nrows, D), jnp.float32),
        mesh=mesh,
        scratch_types=[
            pltpu.VMEM((nch, ch), jnp.int32),
            pltpu.VMEM((nbuf, ch, D), jnp.float32),
            pltpu.SemaphoreType.DMA,
            pltpu.SemaphoreType.DMA,
        ],
    )
    def k(table_hbm, idx_hbm, out_hbm, idx_v, rows_v, gsem, wsem):
        wid = lax.axis_index("s") * 2 + lax.axis_index("c")
        pltpu.sync_copy(idx_hbm.at[wid], idx_v)
        g = [None] * nch
        w = [None] * nch
        for j in range(min(nbuf, nch)):
            g[j] = pltpu.async_copy(table_hbm.at[idx_v.at[j]],
                                    rows_v.at[j % nbuf], gsem)
        for j in range(nch):
            g[j].wait()
            w[j] = pltpu.async_copy(
                rows_v.at[j % nbuf],
                out_hbm.at[pl.ds(wid * bpw + j * ch, ch)], wsem)
            nxt = j + nbuf
            if nxt < nch:
                w[j].wait()
                g[nxt] = pltpu.async_copy(table_hbm.at[idx_v.at[nxt]],
                                          rows_v.at[nxt % nbuf], gsem)
        for j in range(nch):
            if w[j] is not None and (j + nbuf >= nch):
                w[j].wait()

    return k(table, idx.reshape(NW, nch, ch))


def _gemm_body(te_ref, nv_ref, g_ref, w1_ref, b1_ref, w2_ref, b2_ref,
               out_ref):
    t = pl.program_id(0)

    @pl.when(t < nv_ref[0])
    def _():
        g = jnp.dot(g_ref[...].astype(jnp.bfloat16),
                    w1_ref[0].astype(jnp.bfloat16),
                    preferred_element_type=jnp.float32) + b1_ref[0]
        g = 0.5 * g * (1.0 + jax.lax.erf(g * (2.0 ** -0.5)))
        oe = jnp.dot(g.astype(jnp.bfloat16),
                     w2_ref[0].astype(jnp.bfloat16),
                     preferred_element_type=jnp.float32) + b2_ref[0]
        out_ref[...] = oe


def _grouped_gemm(gathered, tile_e, nvalid, W1, b1, W2, b2):
    grid_spec = pltpu.PrefetchScalarGridSpec(
        num_scalar_prefetch=2,
        grid=(NTILES,),
        in_specs=[
            pl.BlockSpec((TMS, D), lambda t, te, nv: (t, 0)),
            pl.BlockSpec((1, D, FF), lambda t, te, nv: (te[t], 0, 0)),
            pl.BlockSpec((1, 1, FF), lambda t, te, nv: (te[t], 0, 0)),
            pl.BlockSpec((1, FF, D), lambda t, te, nv: (te[t], 0, 0)),
            pl.BlockSpec((1, 1, D), lambda t, te, nv: (te[t], 0, 0)),
        ],
        out_specs=pl.BlockSpec((TMS, D), lambda t, te, nv: (t, 0)),
    )
    return pl.pallas_call(
        _gemm_body,
        grid_spec=grid_spec,
        out_shape=jax.ShapeDtypeStruct((PAD, D), jnp.float32),
        compiler_params=pltpu.CompilerParams(
            dimension_semantics=("arbitrary",)),
    )(tile_e, nvalid, gathered, W1.reshape(E, D, FF), b1.reshape(E, 1, FF),
      W2.reshape(E, FF, D), b2.reshape(E, 1, D))


def _comb_body(h1_ref, g0_ref, g1_ref, tw_ref, out_ref):
    tw = tw_ref[...]
    out_ref[...] = (h1_ref[...] + g0_ref[...] * tw[:, 0:1]
                    + g1_ref[...] * tw[:, 1:2])


def _combine(h1, g0, g1, tw):
    tile = pl.BlockSpec((TM, D), lambda i: (i, 0))
    return pl.pallas_call(
        _comb_body,
        grid=(NT,),
        in_specs=[tile, tile, tile, pl.BlockSpec((TM, 2), lambda i: (i, 0))],
        out_specs=tile,
        out_shape=jax.ShapeDtypeStruct((S, D), jnp.float32),
    )(h1, g0, g1, tw)


def _sc_dispatch(hn, sl0, sl1):
    bpw = S // NW
    nch = 2
    ch = bpw // nch
    mesh = plsc.VectorSubcoreMesh(core_axis_name="c", subcore_axis_name="s")

    @functools.partial(
        pl.kernel,
        out_type=jax.ShapeDtypeStruct((PAD, D), jnp.float32),
        mesh=mesh,
        scratch_types=[
            pltpu.VMEM((nch, ch), jnp.int32),
            pltpu.VMEM((nch, ch), jnp.int32),
            pltpu.VMEM((nch, ch, D), jnp.float32),
            pltpu.SemaphoreType.DMA,
            pltpu.SemaphoreType.DMA,
        ],
    )
    def k(hn_hbm, sl0_hbm, sl1_hbm, out_hbm, i0_v, i1_v, rows_v, lsem, ssem):
        wid = lax.axis_index("s") * 2 + lax.axis_index("c")
        pltpu.sync_copy(sl0_hbm.at[wid], i0_v)
        pltpu.sync_copy(sl1_hbm.at[wid], i1_v)
        ld = [None] * nch
        st = [None] * (2 * nch)
        for j in range(nch):
            ld[j] = pltpu.async_copy(
                hn_hbm.at[pl.ds(wid * bpw + j * ch, ch)], rows_v.at[j], lsem)
        for j in range(nch):
            ld[j].wait()
            st[2 * j] = pltpu.async_copy(rows_v.at[j],
                                         out_hbm.at[i0_v.at[j]], ssem)
            st[2 * j + 1] = pltpu.async_copy(rows_v.at[j],
                                             out_hbm.at[i1_v.at[j]], ssem)
        for s_ in st:
            s_.wait()

    return k(hn, sl0.reshape(NW, nch, ch), sl1.reshape(NW, nch, ch))


def _dispatch_indices(rank_enc, counts):
    cnt = counts[0]
    pc = jnp.ceil(cnt / TMS) * TMS
    po = (jnp.cumsum(pc) - pc).astype(jnp.int32)
    ends = (jnp.cumsum(pc)).astype(jnp.int32)
    exp = rank_enc // 65536
    rnk = rank_enc - exp * 65536
    slot = rnk
    for e in range(E):
        slot = slot + jnp.where(exp == e, po[e], 0)
    tile_e = jnp.clip(
        jnp.sum((ends[None, :] <= (jnp.arange(NTILES, dtype=jnp.int32)
                                   * TMS)[:, None]).astype(jnp.int32), axis=1),
        0, E - 1).astype(jnp.int32)
    nvalid = (ends[E - 1] // TMS).reshape(1)
    return slot, tile_e, nvalid


def kernel(x, norm1_w, Wq, bq, Wk, bk, Wv, bv, Wo, bo, norm2_w, Wr, br,
           W1, b1, W2, b2):
    x2 = x.reshape(S, D)
    inv_freq = 1.0 / (10000.0 ** (jnp.arange(0, DH, 2, dtype=jnp.float32) / DH))
    freqs = jnp.arange(S, dtype=jnp.float32)[:, None] * inv_freq[None, :]
    emb = jnp.concatenate([freqs, freqs], axis=-1)
    cosf = jnp.tile(jnp.cos(emb), (1, H))
    sinf = jnp.tile(jnp.sin(emb), (1, H))

    q, k, v = _qkv(x2, norm1_w.reshape(1, D), Wq, Wk, Wv,
                   bq.reshape(1, D), bk.reshape(1, D), bv.reshape(1, D),
                   cosf, sinf)
    o768 = _attn(q, k, v)

    h1, hn, rl, rank_enc, tw, counts = _post(o768, x2, Wo, bo.reshape(1, D),
                                             norm2_w.reshape(1, D), Wr,
                                             br.reshape(1, E))

    slot, tile_e, nvalid = _dispatch_indices(rank_enc, counts)
    gathered = _sc_dispatch(hn, slot[:, 0], slot[:, 1])
    rows = _grouped_gemm(gathered, tile_e, nvalid, W1, b1, W2, b2)
    posf = jnp.concatenate([slot[:, 0], slot[:, 1]])
    g = _sc_gather(rows, posf, 2 * S)
    out = _combine(h1, g[:S], g[S:], tw).reshape(1, S, D)
    return out, rl
